# Initial kernel scaffold; baseline (speedup 1.0000x reference)
import functools

import jax
import jax.numpy as jnp
from jax import lax
from jax.experimental import pallas as pl
from jax.experimental.pallas import tpu as pltpu
from jax.experimental.pallas import tpu_sc as plsc

V = 1000
N = 51200
NC, NS, L = 2, 16, 16
NW = NC * NS
PW = N // NW
CH = 80
NCHUNK = PW // CH


def _lse_body(table_ref, out_ref):
    t = table_ref[...]
    m = jnp.max(t, axis=1)
    s = jnp.sum(jnp.exp(t - m[:, None]), axis=1)
    out_ref[...] = m + jnp.log(s)


def _row_lse(table):
    return pl.pallas_call(
        _lse_body,
        out_shape=jax.ShapeDtypeStruct((V,), jnp.float32),
    )(table)


def _sc_body(idx_hbm, tgt_hbm, table_hbm, lse_hbm, out_hbm, part_hbm,
             idx_v, tgt_v, lse_v, rows_v, acc_v, sem):
    wid = lax.axis_index("s") * NC + lax.axis_index("c")
    base = wid * PW
    pltpu.sync_copy(idx_hbm.at[pl.ds(base, PW)], idx_v)
    pltpu.sync_copy(tgt_hbm.at[pl.ds(base, PW)], tgt_v)
    pltpu.sync_copy(lse_hbm, lse_v)

    def chunk_body(k, acc):
        off = k * CH
        pltpu.async_copy(table_hbm.at[idx_v.at[pl.ds(off, CH)]], rows_v,
                         sem).wait()
        pltpu.sync_copy(rows_v, out_hbm.at[pl.ds(base + off, CH)])
        for i in range(0, CH, L):
            row_ids = lax.iota(jnp.int32, L) + i
            idx16 = idx_v[pl.ds(off + i, L)]
            tgt16 = tgt_v[pl.ds(off + i, L)]
            lse16 = plsc.load_gather(lse_v, [idx16])
            x16 = plsc.load_gather(rows_v, [row_ids, tgt16])
            acc = acc + (lse16 - x16)
        return acc

    acc = lax.fori_loop(0, NCHUNK, chunk_body, jnp.zeros((L,), jnp.float32))
    acc_v[...] = acc
    pltpu.sync_copy(acc_v, part_hbm.at[wid])


def _sc_gather(idx_f, tgt_f, table, lse):
    mesh = plsc.VectorSubcoreMesh(core_axis_name="c", subcore_axis_name="s",
                                  num_cores=NC, num_subcores=NS)
    f = pl.kernel(
        _sc_body,
        out_type=(jax.ShapeDtypeStruct((N, V), jnp.float32),
                  jax.ShapeDtypeStruct((NW, L), jnp.float32)),
        mesh=mesh,
        scratch_types=[
            pltpu.VMEM((PW,), jnp.int32),
            pltpu.VMEM((PW,), jnp.int32),
            pltpu.VMEM((V,), jnp.float32),
            pltpu.VMEM((CH, V), jnp.float32),
            pltpu.VMEM((L,), jnp.float32),
            pltpu.SemaphoreType.DMA,
        ],
    )
    return f(idx_f, tgt_f, table, lse)


def kernel(idx, targets, table):
    idx_f = idx.reshape(-1).astype(jnp.int32)
    tgt_f = targets.reshape(-1).astype(jnp.int32)
    lse = _row_lse(table)
    logits_flat, parts = _sc_gather(idx_f, tgt_f, table, lse)
    logits = logits_flat.reshape(idx.shape[0], idx.shape[1], V)
    loss = parts.sum() / jnp.float32(N)
    return (logits, loss)


# speedup vs baseline: 1.7698x; 1.7698x over previous
import functools

import jax
import jax.numpy as jnp
from jax import lax
from jax.experimental import pallas as pl
from jax.experimental.pallas import tpu as pltpu
from jax.experimental.pallas import tpu_sc as plsc

V = 1000
N = 51200
NC, NS, L = 2, 16, 16
NW = NC * NS
PW = N // NW
CH = 80
NCHUNK = PW // CH


def _lse_body(table_ref, out_ref):
    t = table_ref[...]
    m = jnp.max(t, axis=1)
    s = jnp.sum(jnp.exp(t - m[:, None]), axis=1)
    out_ref[...] = m + jnp.log(s)


def _row_lse(table):
    return pl.pallas_call(
        _lse_body,
        out_shape=jax.ShapeDtypeStruct((V,), jnp.float32),
    )(table)


def _sc_body(idx_hbm, tgt_hbm, table_hbm, lse_hbm, out_hbm, part_hbm,
             idx_v, tgt_v, lse_v, rows_v, acc_v, sem):
    wid = lax.axis_index("s") * NC + lax.axis_index("c")
    base = wid * PW
    pltpu.sync_copy(idx_hbm.at[pl.ds(base, PW)], idx_v)
    pltpu.sync_copy(tgt_hbm.at[pl.ds(base, PW)], tgt_v)
    pltpu.sync_copy(lse_hbm, lse_v)

    def chunk_body(k, acc):
        off = k * CH
        pltpu.async_copy(table_hbm.at[idx_v.at[pl.ds(off, CH)]], rows_v,
                         sem).wait()
        pltpu.sync_copy(rows_v, out_hbm.at[pl.ds(base + off, CH)])
        for i in range(0, CH, L):
            row_ids = lax.iota(jnp.int32, L) + i
            idx16 = idx_v[pl.ds(off + i, L)]
            tgt16 = tgt_v[pl.ds(off + i, L)]
            lse16 = plsc.load_gather(lse_v, [idx16])
            x16 = plsc.load_gather(rows_v, [row_ids, tgt16])
            acc = acc + (lse16 - x16)
        return acc

    acc = lax.fori_loop(0, NCHUNK, chunk_body, jnp.zeros((L,), jnp.float32))
    acc_v[...] = acc
    pltpu.sync_copy(acc_v, part_hbm.at[wid])


def _sc_gather(idx_f, tgt_f, table, lse):
    mesh = plsc.VectorSubcoreMesh(core_axis_name="c", subcore_axis_name="s",
                                  num_cores=NC, num_subcores=NS)
    f = pl.kernel(
        _sc_body,
        out_type=(jax.ShapeDtypeStruct((N, V), jnp.float32),
                  jax.ShapeDtypeStruct((NW, L), jnp.float32)),
        mesh=mesh,
        scratch_types=[
            pltpu.VMEM((PW,), jnp.int32),
            pltpu.VMEM((PW,), jnp.int32),
            pltpu.VMEM((V,), jnp.float32),
            pltpu.VMEM((CH, V), jnp.float32),
            pltpu.VMEM((L,), jnp.float32),
            pltpu.SemaphoreType.DMA,
        ],
        compiler_params=pltpu.CompilerParams(needs_layout_passes=False,
                                             use_tc_tiling_on_sc=False),
    )
    return f(idx_f, tgt_f, table, lse)


def kernel(idx, targets, table):
    idx_f = idx.reshape(-1).astype(jnp.int32)
    tgt_f = targets.reshape(-1).astype(jnp.int32)
    lse = _row_lse(table)
    logits_flat, parts = _sc_gather(idx_f, tgt_f, table, lse)
    logits = logits_flat.reshape(idx.shape[0], idx.shape[1], V)
    loss = parts.sum() / jnp.float32(N)
    return (logits, loss)
